# baseline (speedup 1.0000x reference)
import functools
import math

import jax
import jax.numpy as jnp
from jax import lax
from jax.experimental import pallas as pl
from jax.experimental.pallas import tpu as pltpu
from jax.experimental.pallas import tpu_sc as plsc

HID = 64
EMB = 128

_info = plsc.get_sparse_core_info()
_NC, _NS = _info.num_cores, _info.num_subcores
_NW = _NC * _NS

_CHUNK = 128


def _proj_body(tt_ref, w_ref, b_ref, o_ref):
    g = tt_ref[...]
    h = 0.5 * g * (1.0 + lax.erf(g * (1.0 / math.sqrt(2.0))))
    acc = lax.dot_general(
        h, w_ref[...], (((0,), (0,)), ((), ())),
        preferred_element_type=jnp.float32,
    )
    o_ref[...] = acc + b_ref[...]


def _project_table(table_t, w_t, b):
    v = table_t.shape[1]
    c = 32768
    grid = (pl.cdiv(v, c),)
    return pl.pallas_call(
        _proj_body,
        grid=grid,
        in_specs=[
            pl.BlockSpec((HID, c), lambda i: (0, i)),
            pl.BlockSpec((HID, EMB), lambda i: (0, 0)),
            pl.BlockSpec((1, EMB), lambda i: (0, 0)),
        ],
        out_specs=pl.BlockSpec((c, EMB), lambda i: (i, 0)),
        out_shape=jax.ShapeDtypeStruct((v, EMB), jnp.float32),
    )(table_t, w_t, b.reshape(1, EMB))


_NSTREAM = 2
_STEP = _NSTREAM * _CHUNK


def _sc_gather(proj, idx_flat):
    n = idx_flat.shape[0]
    assert n % (_NW * _STEP) == 0
    per_w = n // _NW
    chunks = per_w // _STEP
    assert chunks % 2 == 0
    half = chunks // 2
    mesh = plsc.VectorSubcoreMesh(core_axis_name="c", subcore_axis_name="s")

    @functools.partial(
        pl.kernel,
        mesh=mesh,
        out_type=jax.ShapeDtypeStruct((n, EMB), jnp.float32),
        scratch_types=[
            pltpu.VMEM((per_w,), jnp.int32),
            pltpu.VMEM((_STEP, EMB), jnp.float32),
            pltpu.VMEM((_STEP, EMB), jnp.float32),
            pltpu.SemaphoreType.DMA,
            pltpu.SemaphoreType.DMA,
        ],
        compiler_params=pltpu.CompilerParams(use_tc_tiling_on_sc=True),
    )
    def k(proj_hbm, idx_hbm, out_hbm, idx_all, rows_a, rows_b, sem_a, sem_b):
        wid = lax.axis_index("s") * _NC + lax.axis_index("c")
        base = wid * per_w
        pltpu.sync_copy(idx_hbm.at[pl.ds(base, per_w)], idx_all)

        def gathers(chunk_i, rows_v, sem):
            return [
                pltpu.make_async_copy(
                    proj_hbm.at[
                        idx_all.at[pl.ds(chunk_i * _STEP + j * _CHUNK, _CHUNK)]
                    ],
                    rows_v.at[pl.ds(j * _CHUNK, _CHUNK)],
                    sem,
                )
                for j in range(_NSTREAM)
            ]

        def fire(chunk_i, rows_v, sem):
            for g in gathers(chunk_i, rows_v, sem):
                g.start()

        def drain(chunk_i, rows_v, sem):
            for g in gathers(chunk_i, rows_v, sem):
                g.wait()
            pltpu.sync_copy(rows_v, out_hbm.at[pl.ds(base + chunk_i * _STEP, _STEP)])

        fire(0, rows_a, sem_a)

        def body(i, carry):
            fire(2 * i + 1, rows_b, sem_b)
            drain(2 * i, rows_a, sem_a)

            @pl.when(i < half - 1)
            def _():
                fire(2 * i + 2, rows_a, sem_a)

            drain(2 * i + 1, rows_b, sem_b)
            return carry

        lax.fori_loop(0, half, body, jnp.int32(0))

    return k(proj, idx_flat)


def kernel(x, table, W_proj, b_proj):
    bsz, seq = x.shape
    idx_flat = x.reshape(-1).astype(jnp.int32)
    table_t = jnp.swapaxes(table, 0, 1)
    w_t = jnp.swapaxes(W_proj, 0, 1)
    proj = _project_table(table_t, w_t, b_proj)
    out = _sc_gather(proj, idx_flat)
    return out.reshape(bsz, seq, EMB)


# speedup vs baseline: 1.0040x; 1.0040x over previous
import functools
import math

import jax
import jax.numpy as jnp
from jax import lax
from jax.experimental import pallas as pl
from jax.experimental.pallas import tpu as pltpu
from jax.experimental.pallas import tpu_sc as plsc

HID = 64
EMB = 128

_info = plsc.get_sparse_core_info()
_NC, _NS = _info.num_cores, _info.num_subcores
_NW = _NC * _NS

_CHUNK = 128


def _proj_body(tt_ref, w_ref, b_ref, o_ref):
    g = tt_ref[...]
    h = 0.5 * g * (1.0 + lax.erf(g * (1.0 / math.sqrt(2.0))))
    acc = lax.dot_general(
        h, w_ref[...], (((0,), (0,)), ((), ())),
        preferred_element_type=jnp.float32,
    )
    o_ref[...] = acc + b_ref[...]


def _project_table(table_t, w_t, b):
    v = table_t.shape[1]
    c = 40960
    grid = (pl.cdiv(v, c),)
    return pl.pallas_call(
        _proj_body,
        grid=grid,
        compiler_params=pltpu.CompilerParams(vmem_limit_bytes=63 * 1024 * 1024),
        in_specs=[
            pl.BlockSpec((HID, c), lambda i: (0, i)),
            pl.BlockSpec((HID, EMB), lambda i: (0, 0)),
            pl.BlockSpec((1, EMB), lambda i: (0, 0)),
        ],
        out_specs=pl.BlockSpec((c, EMB), lambda i: (i, 0)),
        out_shape=jax.ShapeDtypeStruct((v, EMB), jnp.float32),
    )(table_t, w_t, b.reshape(1, EMB))


_NSTREAM = 2
_STEP = _NSTREAM * _CHUNK


def _sc_gather(proj, idx_flat):
    n = idx_flat.shape[0]
    assert n % (_NW * _STEP) == 0
    per_w = n // _NW
    chunks = per_w // _STEP
    assert chunks % 2 == 0
    half = chunks // 2
    mesh = plsc.VectorSubcoreMesh(core_axis_name="c", subcore_axis_name="s")

    @functools.partial(
        pl.kernel,
        mesh=mesh,
        out_type=jax.ShapeDtypeStruct((n, EMB), jnp.float32),
        scratch_types=[
            pltpu.VMEM((per_w,), jnp.int32),
            pltpu.VMEM((_STEP, EMB), jnp.float32),
            pltpu.VMEM((_STEP, EMB), jnp.float32),
            pltpu.SemaphoreType.DMA,
            pltpu.SemaphoreType.DMA,
        ],
        compiler_params=pltpu.CompilerParams(use_tc_tiling_on_sc=True),
    )
    def k(proj_hbm, idx_hbm, out_hbm, idx_all, rows_a, rows_b, sem_a, sem_b):
        wid = lax.axis_index("s") * _NC + lax.axis_index("c")
        base = wid * per_w
        pltpu.sync_copy(idx_hbm.at[pl.ds(base, per_w)], idx_all)

        def gathers(chunk_i, rows_v, sem):
            return [
                pltpu.make_async_copy(
                    proj_hbm.at[
                        idx_all.at[pl.ds(chunk_i * _STEP + j * _CHUNK, _CHUNK)]
                    ],
                    rows_v.at[pl.ds(j * _CHUNK, _CHUNK)],
                    sem,
                )
                for j in range(_NSTREAM)
            ]

        def fire(chunk_i, rows_v, sem):
            for g in gathers(chunk_i, rows_v, sem):
                g.start()

        def drain(chunk_i, rows_v, sem):
            for g in gathers(chunk_i, rows_v, sem):
                g.wait()
            pltpu.sync_copy(rows_v, out_hbm.at[pl.ds(base + chunk_i * _STEP, _STEP)])

        fire(0, rows_a, sem_a)

        def body(i, carry):
            fire(2 * i + 1, rows_b, sem_b)
            drain(2 * i, rows_a, sem_a)

            @pl.when(i < half - 1)
            def _():
                fire(2 * i + 2, rows_a, sem_a)

            drain(2 * i + 1, rows_b, sem_b)
            return carry

        lax.fori_loop(0, half, body, jnp.int32(0))

    return k(proj, idx_flat)


def kernel(x, table, W_proj, b_proj):
    bsz, seq = x.shape
    idx_flat = x.reshape(-1).astype(jnp.int32)
    table_t = jnp.swapaxes(table, 0, 1)
    w_t = jnp.swapaxes(W_proj, 0, 1)
    proj = _project_table(table_t, w_t, b_proj)
    out = _sc_gather(proj, idx_flat)
    return out.reshape(bsz, seq, EMB)
